# baseline (speedup 1.0000x reference)
import functools

import jax
import jax.numpy as jnp
from jax import lax
from jax.experimental import pallas as pl
from jax.experimental.pallas import tpu as pltpu
from jax.experimental.pallas import tpu_sc as plsc

NC = 2
NS = 16
NW = NC * NS
GROUP = 128
NBUF = 4


@functools.partial(jax.jit, static_argnums=(2, 3))
def _emb_lookup(ids2d, table, n_rows, dim):
    rows_per_w = n_rows // NW
    n_g = rows_per_w // GROUP
    mesh = plsc.VectorSubcoreMesh(core_axis_name="c", subcore_axis_name="s")

    @functools.partial(
        pl.kernel,
        out_type=jax.ShapeDtypeStruct((n_rows, dim), jnp.float32),
        mesh=mesh,
        scratch_types=[
            pltpu.VMEM((n_g, GROUP), jnp.int32),
            [pltpu.VMEM((GROUP, dim), jnp.float32) for _ in range(NBUF)],
            [pltpu.SemaphoreType.DMA for _ in range(NBUF)],
        ],
        compiler_params=pltpu.CompilerParams(use_tc_tiling_on_sc=False),
    )
    def k(ids_hbm, table_hbm, out_hbm, idx_v, bufs, sems):
        wid = lax.axis_index("s") * NC + lax.axis_index("c")
        base_g = wid * n_g
        base_row = base_g * GROUP

        pltpu.sync_copy(ids_hbm.at[pl.ds(base_g, n_g)], idx_v)

        def start(g, b):
            pltpu.async_copy(table_hbm.at[idx_v.at[g]], bufs[b], sems[b])

        def finish(g, b):
            pltpu.make_async_copy(
                table_hbm.at[idx_v.at[g]], bufs[b], sems[b]
            ).wait()
            pltpu.sync_copy(
                bufs[b], out_hbm.at[pl.ds(base_row + g * GROUP, GROUP)]
            )

        for b in range(NBUF):
            start(b, b)

        def outer(go, _):
            for b in range(NBUF):
                g = go * NBUF + b
                finish(g, b)
                start(g + NBUF, b)
            return _

        lax.fori_loop(0, n_g // NBUF - 1, outer, 0, unroll=False)

        for b in range(NBUF):
            g = n_g - NBUF + b
            finish(g, b)

    return k(ids2d, table)


def kernel(input_ids, weight):
    batch, seq = input_ids.shape
    vocab, dim = weight.shape
    n_rows = batch * seq
    assert n_rows % (NW * GROUP) == 0 and dim % 16 == 0
    ids2d = input_ids.astype(jnp.int32).reshape(n_rows // GROUP, GROUP)
    out = _emb_lookup(ids2d, weight, n_rows, dim)
    return out.reshape(batch, seq, dim)
